# Initial kernel scaffold; baseline (speedup 1.0000x reference)
import functools

import jax
import jax.numpy as jnp
from jax import lax
from jax.experimental import pallas as pl
from jax.experimental.pallas import tpu as pltpu
from jax.experimental.pallas import tpu_sc as plsc

_N = 50000
_E = 800000
_D = 64
_G = 128
_H = _D // 2

_BLK = 512
_GRID = 98
_NR = _GRID * _BLK

_NT = 16
_NCORE = 2
_EC = 128

_EP = _E + _N
_EPAD = 851968
_ECH16 = _EPAD // (_NT * _EC)
_ECH32 = _EPAD // (2 * _NT * _EC)
_EBLK = 52

_NP2 = 53248
_NCH = _NP2 // (2 * _NT * _EC)
_ZR = _NR // _NT

_GP = 256

_mesh = plsc.VectorSubcoreMesh(core_axis_name="c", subcore_axis_name="s")
_f32 = jnp.float32
_i32 = jnp.int32


def _fill_ones(ref, n):
    for i in range(n // 16):
        ref[pl.ds(i * 16, 16)] = jnp.ones((16,), _f32)


@functools.partial(
    pl.kernel,
    out_type=[
        jax.ShapeDtypeStruct((_NP2, _D), _f32),
        jax.ShapeDtypeStruct((_NCORE, _NR), _f32),
    ],
    mesh=_mesh,
    scratch_types=[
        pltpu.VMEM((_NCH, _EC), _i32),
        pltpu.VMEM((_ECH32, _EC), _i32),
        pltpu.VMEM((_EC, _D), _f32),
        pltpu.VMEM((_EC,), _f32),
        pltpu.VMEM_SHARED((_NR,), _f32),
        pltpu.SemaphoreType.DMA,
    ],
)
def _sc_prep(emb_h, xp_h, dst32_h, z1_h, h0_h, degp_h,
             xbuf, dbuf, rbuf, ones_v, deg_s, sem):
    c = lax.axis_index("c")
    s = lax.axis_index("s")
    wid = s * _NCORE + c

    pltpu.sync_copy(z1_h.at[pl.ds(0, _ZR)], deg_s.at[pl.ds(s * _ZR, _ZR)])
    _fill_ones(ones_v, _EC)

    pltpu.sync_copy(xp_h.at[wid], xbuf)
    base = wid * _NCH * _EC
    for j in range(_NCH):
        pltpu.async_copy(emb_h.at[xbuf.at[j]], rbuf, sem).wait()
        pltpu.sync_copy(rbuf, h0_h.at[pl.ds(base + j * _EC, _EC)])

    pltpu.sync_copy(dst32_h.at[wid], dbuf)
    plsc.subcore_barrier()

    def deg_body(j, carry):
        pltpu.sync_copy(ones_v, deg_s.at[dbuf.at[j]], add=True)
        return carry

    lax.fori_loop(0, _ECH32, deg_body, 0)
    plsc.subcore_barrier()
    pltpu.sync_copy(deg_s.at[pl.ds(s * _ZR, _ZR)],
                    degp_h.at[c, pl.ds(s * _ZR, _ZR)])


@functools.partial(
    pl.kernel,
    out_type=[
        jax.ShapeDtypeStruct((_NR, _H), _f32),
        jax.ShapeDtypeStruct((_NR, _H), _f32),
    ],
    mesh=_mesh,
    scratch_types=[
        pltpu.VMEM((_EBLK, _EC), _i32),
        pltpu.VMEM((_EBLK, _EC), _i32),
        pltpu.VMEM((_EC, _H), _f32),
        pltpu.VMEM_SHARED((_NR, _H), _f32),
        pltpu.SemaphoreType.DMA,
    ],
)
def _sc_edge(ga_h, gb_h, src16_h, dst16_h, z2_h, outa_h, outb_h,
             sbuf, dbuf, rbuf, acc_s, sem):
    c = lax.axis_index("c")
    s = lax.axis_index("s")

    pltpu.sync_copy(z2_h.at[pl.ds(0, _ZR)], acc_s.at[pl.ds(s * _ZR, _ZR)])
    plsc.subcore_barrier()

    def run(g_h, out_h):
        for blk in range(_ECH16 // _EBLK):
            pltpu.sync_copy(src16_h.at[s, pl.ds(blk * _EBLK, _EBLK)], sbuf)
            pltpu.sync_copy(dst16_h.at[s, pl.ds(blk * _EBLK, _EBLK)], dbuf)

            def body(j, carry):
                pltpu.async_copy(g_h.at[sbuf.at[j]], rbuf, sem).wait()
                pltpu.sync_copy(rbuf, acc_s.at[dbuf.at[j]], add=True)
                return carry

            lax.fori_loop(0, _EBLK, body, 0)
        plsc.subcore_barrier()
        pltpu.sync_copy(acc_s.at[pl.ds(s * _ZR, _ZR)],
                        out_h.at[pl.ds(s * _ZR, _ZR)])

    @pl.when(c == 0)
    def _():
        run(ga_h, outa_h)

    @pl.when(c == 1)
    def _():
        run(gb_h, outb_h)


@functools.partial(
    pl.kernel,
    out_type=[
        jax.ShapeDtypeStruct((_NCORE, _GP, _D), _f32),
        jax.ShapeDtypeStruct((_NCORE, _GP), _f32),
    ],
    mesh=_mesh,
    scratch_types=[
        pltpu.VMEM((_NCH, _EC), _i32),
        pltpu.VMEM((_NCH, _EC), _i32),
        pltpu.VMEM((_EC, _D), _f32),
        pltpu.VMEM((_EC,), _f32),
        pltpu.VMEM_SHARED((_GP, _D), _f32),
        pltpu.VMEM_SHARED((_GP,), _f32),
        pltpu.SemaphoreType.DMA,
    ],
)
def _sc_pool(h2_h, idp_h, bp_h, z1_h, sums_h, cnts_h,
             ibuf, bbuf, rbuf, ones_v, sums_s, cnts_s, sem):
    c = lax.axis_index("c")
    s = lax.axis_index("s")
    wid = s * _NCORE + c
    rows = _GP // _NT

    for r in range(rows):
        for k in range(_D // 16):
            rbuf[r, pl.ds(k * 16, 16)] = jnp.zeros((16,), _f32)
    pltpu.sync_copy(rbuf.at[pl.ds(0, rows)], sums_s.at[pl.ds(s * rows, rows)])
    ones_v[pl.ds(0, 16)] = jnp.zeros((16,), _f32)
    pltpu.sync_copy(ones_v.at[pl.ds(0, rows)],
                    cnts_s.at[pl.ds(s * rows, rows)])
    _fill_ones(ones_v, _EC)
    plsc.subcore_barrier()

    pltpu.sync_copy(idp_h.at[wid], ibuf)
    pltpu.sync_copy(bp_h.at[wid], bbuf)

    def body(j, carry):
        pltpu.async_copy(h2_h.at[ibuf.at[j]], rbuf, sem).wait()
        pltpu.sync_copy(rbuf, sums_s.at[bbuf.at[j]], add=True)
        pltpu.sync_copy(ones_v, cnts_s.at[bbuf.at[j]], add=True)
        return carry

    lax.fori_loop(0, _NCH, body, 0)
    plsc.subcore_barrier()
    pltpu.sync_copy(sums_s.at[pl.ds(s * rows, rows)],
                    sums_h.at[c, pl.ds(s * rows, rows)])
    pltpu.sync_copy(cnts_s.at[pl.ds(s * rows, rows)],
                    cnts_h.at[c, pl.ds(s * rows, rows)])


def _dinv_of(degp_b):
    deg = degp_b[0, :] + degp_b[1, :]
    return lax.rsqrt(jnp.maximum(deg, 1.0))


def _tc_g1_body(h0_b, degp_b, w1_b, ga_b, gb_b):
    dinv = _dinv_of(degp_b)
    g = jnp.dot(h0_b[...], w1_b[...],
                preferred_element_type=_f32) * dinv[:, None]
    ga_b[...] = g[:, :_H]
    gb_b[...] = g[:, _H:]


def _tc_mid_body(aa_b, ab_b, degp_b, b1_b, w2_b, ga_b, gb_b):
    dinv = _dinv_of(degp_b)
    agg = jnp.concatenate([aa_b[...], ab_b[...]], axis=1)
    h1 = jnp.maximum(agg * dinv[:, None] + b1_b[...], 0.0)
    g2 = jnp.dot(h1, w2_b[...], preferred_element_type=_f32) * dinv[:, None]
    ga_b[...] = g2[:, :_H]
    gb_b[...] = g2[:, _H:]


def _tc_h2_body(aa_b, ab_b, degp_b, b2_b, h2_b):
    dinv = _dinv_of(degp_b)
    agg = jnp.concatenate([aa_b[...], ab_b[...]], axis=1)
    h2_b[...] = jnp.maximum(agg * dinv[:, None] + b2_b[...], 0.0)


def _tc_out_body(sp_b, cp_b, wc_b, bc_b, o_b):
    sums = sp_b[0] + sp_b[1]
    cnts = cp_b[0, :] + cp_b[1, :]
    pooled = sums[:_G] / jnp.maximum(cnts[:_G], 1.0)[:, None]
    o_b[...] = jnp.dot(pooled, wc_b[...],
                       preferred_element_type=_f32) + bc_b[...]


def _row_spec(w):
    return pl.BlockSpec((_BLK, w), lambda i: (i, 0))


def _full_spec(shape):
    nd = len(shape)
    return pl.BlockSpec(shape, lambda i, _nd=nd: (0,) * _nd)


def _tc_g1(h0, degp, W1):
    return pl.pallas_call(
        _tc_g1_body,
        grid=(_GRID,),
        in_specs=[_row_spec(_D),
                  pl.BlockSpec((2, _BLK), lambda i: (0, i)),
                  _full_spec((_D, _D))],
        out_specs=[_row_spec(_H), _row_spec(_H)],
        out_shape=[jax.ShapeDtypeStruct((_NR, _H), _f32)] * 2,
    )(h0, degp, W1)


def _tc_mid(aa, ab, degp, b1r, W2):
    return pl.pallas_call(
        _tc_mid_body,
        grid=(_GRID,),
        in_specs=[_row_spec(_H), _row_spec(_H),
                  pl.BlockSpec((2, _BLK), lambda i: (0, i)),
                  _full_spec((1, _D)), _full_spec((_D, _D))],
        out_specs=[_row_spec(_H), _row_spec(_H)],
        out_shape=[jax.ShapeDtypeStruct((_NR, _H), _f32)] * 2,
    )(aa, ab, degp, b1r, W2)


def _tc_h2(aa, ab, degp, b2r):
    return pl.pallas_call(
        _tc_h2_body,
        grid=(_GRID,),
        in_specs=[_row_spec(_H), _row_spec(_H),
                  pl.BlockSpec((2, _BLK), lambda i: (0, i)),
                  _full_spec((1, _D))],
        out_specs=_row_spec(_D),
        out_shape=jax.ShapeDtypeStruct((_NR, _D), _f32),
    )(aa, ab, degp, b2r)


def _tc_out(sums_p, cnts_p, Wc, bcr):
    return pl.pallas_call(
        _tc_out_body,
        out_shape=jax.ShapeDtypeStruct((_G, 2), _f32),
    )(sums_p, cnts_p, Wc, bcr)


def kernel(x, edge_index, batch, emb, W1, b1, W2, b2, Wc, bc):
    loop = jnp.arange(_N, dtype=_i32)
    pad = _EPAD - _EP
    src = jnp.concatenate([edge_index[0], loop,
                           jnp.zeros((pad,), _i32)])
    dst = jnp.concatenate([edge_index[1], loop,
                           jnp.full((pad,), _N, _i32)])
    src16 = src.reshape(_NT, _ECH16, _EC)
    dst16 = dst.reshape(_NT, _ECH16, _EC)
    dst32 = dst.reshape(2 * _NT, _ECH32, _EC)

    npad = _NP2 - _N
    xp = jnp.concatenate([x, jnp.zeros((npad,), _i32)]).reshape(
        2 * _NT, _NCH, _EC)
    idp = jnp.concatenate([loop, jnp.zeros((npad,), _i32)]).reshape(
        2 * _NT, _NCH, _EC)
    bp = jnp.concatenate([batch, jnp.full((npad,), _G, _i32)]).reshape(
        2 * _NT, _NCH, _EC)

    z1 = jnp.zeros((_NR,), _f32)
    z2 = jnp.zeros((_NR, _H), _f32)
    b1r = b1.reshape(1, _D)
    b2r = b2.reshape(1, _D)
    bcr = bc.reshape(1, 2)

    h0, degp = _sc_prep(emb, xp, dst32, z1)
    g1a, g1b = _tc_g1(h0, degp, W1)
    a1a, a1b = _sc_edge(g1a, g1b, src16, dst16, z2)
    g2a, g2b = _tc_mid(a1a, a1b, degp, b1r, W2)
    a2a, a2b = _sc_edge(g2a, g2b, src16, dst16, z2)
    h2 = _tc_h2(a2a, a2b, degp, b2r)
    sums_p, cnts_p = _sc_pool(h2, idp, bp, z1)
    return _tc_out(sums_p, cnts_p, Wc, bcr)


# speedup vs baseline: 15.7641x; 15.7641x over previous
import functools

import jax
import jax.numpy as jnp
from jax import lax
from jax.experimental import pallas as pl
from jax.experimental.pallas import tpu as pltpu
from jax.experimental.pallas import tpu_sc as plsc

_N = 50000
_E = 800000
_D = 64
_G = 128
_H = _D // 2

_BLK = 512
_GRID = 98
_NR = _GRID * _BLK

_NT = 16
_NCORE = 2
_EC = 128

_EP = _E + _N
_EPAD = 851968
_ECH16 = _EPAD // (_NT * _EC)
_ECH32 = _EPAD // (2 * _NT * _EC)
_EBLK = 104

_NP2 = 53248
_NCH = _NP2 // (2 * _NT * _EC)
_ZR = _NR // _NT

_GP = 256

_mesh = plsc.VectorSubcoreMesh(core_axis_name="c", subcore_axis_name="s")
_f32 = jnp.float32
_i32 = jnp.int32


def _fill_ones(ref, n):
    for i in range(n // 16):
        ref[pl.ds(i * 16, 16)] = jnp.ones((16,), _f32)


@functools.partial(
    pl.kernel,
    out_type=[
        jax.ShapeDtypeStruct((_NP2, _D), _f32),
        jax.ShapeDtypeStruct((_NR,), _f32),
        jax.ShapeDtypeStruct((_NR,), _f32),
    ],
    mesh=_mesh,
    compiler_params=pltpu.CompilerParams(use_tc_tiling_on_sc=False),
    scratch_types=[
        pltpu.VMEM((_NCH, _EC), _i32),
        pltpu.VMEM((_ECH32, _EC), _i32),
        pltpu.VMEM((_EC, _D), _f32),
        pltpu.VMEM((_EC,), _f32),
        pltpu.VMEM_SHARED((_NR,), _f32),
        pltpu.SemaphoreType.DMA,
    ],
)
def _sc_prep(emb_h, xp_h, dst32_h, z1_h, h0_h, dega_h, degb_h,
             xbuf, dbuf, rbuf, ones_v, deg_s, sem):
    c = lax.axis_index("c")
    s = lax.axis_index("s")
    wid = s * _NCORE + c

    @pl.when(s == 0)
    def _():
        pltpu.sync_copy(z1_h, deg_s)

    _fill_ones(ones_v, _EC)

    pltpu.sync_copy(xp_h.at[wid], xbuf)
    base = wid * _NCH * _EC
    for j in range(_NCH):
        pltpu.async_copy(emb_h.at[xbuf.at[j]], rbuf, sem).wait()
        pltpu.sync_copy(rbuf, h0_h.at[pl.ds(base + j * _EC, _EC)])

    pltpu.sync_copy(dst32_h.at[wid], dbuf)
    plsc.subcore_barrier()

    def deg_body(j, carry):
        pltpu.sync_copy(ones_v, deg_s.at[dbuf.at[j]], add=True)
        return carry

    lax.fori_loop(0, _ECH32, deg_body, 0)
    plsc.subcore_barrier()

    @pl.when(jnp.logical_and(s == 0, c == 0))
    def _():
        pltpu.sync_copy(deg_s, dega_h)

    @pl.when(jnp.logical_and(s == 0, c == 1))
    def _():
        pltpu.sync_copy(deg_s, degb_h)


@functools.partial(
    pl.kernel,
    out_type=[
        jax.ShapeDtypeStruct((_NR, _H), _f32),
        jax.ShapeDtypeStruct((_NR, _H), _f32),
    ],
    mesh=_mesh,
    compiler_params=pltpu.CompilerParams(use_tc_tiling_on_sc=False),
    scratch_types=[
        pltpu.VMEM((_EBLK, _EC), _i32),
        pltpu.VMEM((_EBLK, _EC), _i32),
        pltpu.VMEM((_EC, _H), _f32),
        pltpu.VMEM_SHARED((_NR, _H), _f32),
        pltpu.SemaphoreType.DMA,
    ],
)
def _sc_edge(ga_h, gb_h, src16_h, dst16_h, z2_h, outa_h, outb_h,
             sbuf, dbuf, rbuf, acc_s, sem):
    c = lax.axis_index("c")
    s = lax.axis_index("s")

    pltpu.sync_copy(z2_h.at[pl.ds(0, _ZR)], acc_s.at[pl.ds(s * _ZR, _ZR)])
    plsc.subcore_barrier()

    def run(g_h, out_h):
        for blk in range(_ECH16 // _EBLK):
            pltpu.sync_copy(src16_h.at[s, pl.ds(blk * _EBLK, _EBLK)], sbuf)
            pltpu.sync_copy(dst16_h.at[s, pl.ds(blk * _EBLK, _EBLK)], dbuf)

            def body(j, carry):
                pltpu.async_copy(g_h.at[sbuf.at[j]], rbuf, sem).wait()
                pltpu.sync_copy(rbuf, acc_s.at[dbuf.at[j]], add=True)
                return carry

            lax.fori_loop(0, _EBLK, body, 0)
        plsc.subcore_barrier()
        pltpu.sync_copy(acc_s.at[pl.ds(s * _ZR, _ZR)],
                        out_h.at[pl.ds(s * _ZR, _ZR)])

    @pl.when(c == 0)
    def _():
        run(ga_h, outa_h)

    @pl.when(c == 1)
    def _():
        run(gb_h, outb_h)


@functools.partial(
    pl.kernel,
    out_type=[
        jax.ShapeDtypeStruct((_GP, _D), _f32),
        jax.ShapeDtypeStruct((_GP, _D), _f32),
        jax.ShapeDtypeStruct((_GP,), _f32),
        jax.ShapeDtypeStruct((_GP,), _f32),
    ],
    mesh=_mesh,
    compiler_params=pltpu.CompilerParams(use_tc_tiling_on_sc=False),
    scratch_types=[
        pltpu.VMEM((_NCH, _EC), _i32),
        pltpu.VMEM((_NCH, _EC), _i32),
        pltpu.VMEM((_EC, _D), _f32),
        pltpu.VMEM((_EC,), _f32),
        pltpu.VMEM_SHARED((_GP, _D), _f32),
        pltpu.VMEM_SHARED((_GP,), _f32),
        pltpu.SemaphoreType.DMA,
    ],
)
def _sc_pool(h2_h, idp_h, bp_h, z1_h, sumsa_h, sumsb_h, cntsa_h, cntsb_h,
             ibuf, bbuf, rbuf, ones_v, sums_s, cnts_s, sem):
    c = lax.axis_index("c")
    s = lax.axis_index("s")
    wid = s * _NCORE + c
    rows = _GP // _NT

    for r in range(rows):
        for k in range(_D // 16):
            rbuf[r, pl.ds(k * 16, 16)] = jnp.zeros((16,), _f32)
    pltpu.sync_copy(rbuf.at[pl.ds(0, rows)], sums_s.at[pl.ds(s * rows, rows)])
    ones_v[pl.ds(0, 16)] = jnp.zeros((16,), _f32)
    pltpu.sync_copy(ones_v.at[pl.ds(0, rows)],
                    cnts_s.at[pl.ds(s * rows, rows)])
    _fill_ones(ones_v, _EC)
    plsc.subcore_barrier()

    pltpu.sync_copy(idp_h.at[wid], ibuf)
    pltpu.sync_copy(bp_h.at[wid], bbuf)

    def body(j, carry):
        pltpu.async_copy(h2_h.at[ibuf.at[j]], rbuf, sem).wait()
        pltpu.sync_copy(rbuf, sums_s.at[bbuf.at[j]], add=True)
        pltpu.sync_copy(ones_v, cnts_s.at[bbuf.at[j]], add=True)
        return carry

    lax.fori_loop(0, _NCH, body, 0)
    plsc.subcore_barrier()

    @pl.when(jnp.logical_and(s == 0, c == 0))
    def _():
        pltpu.sync_copy(sums_s, sumsa_h)
        pltpu.sync_copy(cnts_s, cntsa_h)

    @pl.when(jnp.logical_and(s == 0, c == 1))
    def _():
        pltpu.sync_copy(sums_s, sumsb_h)
        pltpu.sync_copy(cnts_s, cntsb_h)


def _dinv_of(da_b, db_b):
    deg = da_b[0, :] + db_b[0, :]
    return lax.rsqrt(jnp.maximum(deg, 1.0))


def _tc_g1_body(h0_b, da_b, db_b, w1_b, ga_b, gb_b):
    dinv = _dinv_of(da_b, db_b)
    g = jnp.dot(h0_b[...], w1_b[...],
                preferred_element_type=_f32) * dinv[:, None]
    ga_b[...] = g[:, :_H]
    gb_b[...] = g[:, _H:]


def _tc_mid_body(aa_b, ab_b, da_b, db_b, b1_b, w2_b, ga_b, gb_b):
    dinv = _dinv_of(da_b, db_b)
    agg = jnp.concatenate([aa_b[...], ab_b[...]], axis=1)
    h1 = jnp.maximum(agg * dinv[:, None] + b1_b[...], 0.0)
    g2 = jnp.dot(h1, w2_b[...], preferred_element_type=_f32) * dinv[:, None]
    ga_b[...] = g2[:, :_H]
    gb_b[...] = g2[:, _H:]


def _tc_h2_body(aa_b, ab_b, da_b, db_b, b2_b, h2_b):
    dinv = _dinv_of(da_b, db_b)
    agg = jnp.concatenate([aa_b[...], ab_b[...]], axis=1)
    h2_b[...] = jnp.maximum(agg * dinv[:, None] + b2_b[...], 0.0)


def _tc_out_body(sa_b, sb_b, ca_b, cb_b, wc_b, bc_b, o_b):
    sums = sa_b[...] + sb_b[...]
    cnts = ca_b[0, :] + cb_b[0, :]
    pooled = sums[:_G] / jnp.maximum(cnts[:_G], 1.0)[:, None]
    o_b[...] = jnp.dot(pooled, wc_b[...],
                       preferred_element_type=_f32) + bc_b[...]


def _row_spec(w):
    return pl.BlockSpec((_BLK, w), lambda i: (i, 0))


def _full_spec(shape):
    nd = len(shape)
    return pl.BlockSpec(shape, lambda i, _nd=nd: (0,) * _nd)


_deg_spec = pl.BlockSpec((1, _BLK), lambda i: (0, i))


def _tc_g1(h0, dega, degb, W1):
    return pl.pallas_call(
        _tc_g1_body,
        grid=(_GRID,),
        in_specs=[_row_spec(_D), _deg_spec, _deg_spec,
                  _full_spec((_D, _D))],
        out_specs=[_row_spec(_H), _row_spec(_H)],
        out_shape=[jax.ShapeDtypeStruct((_NR, _H), _f32)] * 2,
    )(h0, dega, degb, W1)


def _tc_mid(aa, ab, dega, degb, b1r, W2):
    return pl.pallas_call(
        _tc_mid_body,
        grid=(_GRID,),
        in_specs=[_row_spec(_H), _row_spec(_H), _deg_spec, _deg_spec,
                  _full_spec((1, _D)), _full_spec((_D, _D))],
        out_specs=[_row_spec(_H), _row_spec(_H)],
        out_shape=[jax.ShapeDtypeStruct((_NR, _H), _f32)] * 2,
    )(aa, ab, dega, degb, b1r, W2)


def _tc_h2(aa, ab, dega, degb, b2r):
    return pl.pallas_call(
        _tc_h2_body,
        grid=(_GRID,),
        in_specs=[_row_spec(_H), _row_spec(_H), _deg_spec, _deg_spec,
                  _full_spec((1, _D))],
        out_specs=_row_spec(_D),
        out_shape=jax.ShapeDtypeStruct((_NR, _D), _f32),
    )(aa, ab, dega, degb, b2r)


def _tc_out(sumsa, sumsb, cntsa, cntsb, Wc, bcr):
    return pl.pallas_call(
        _tc_out_body,
        out_shape=jax.ShapeDtypeStruct((_G, 2), _f32),
    )(sumsa, sumsb, cntsa, cntsb, Wc, bcr)


def kernel(x, edge_index, batch, emb, W1, b1, W2, b2, Wc, bc):
    loop = jnp.arange(_N, dtype=_i32)
    pad = _EPAD - _EP
    src = jnp.concatenate([edge_index[0], loop,
                           jnp.zeros((pad,), _i32)])
    dst = jnp.concatenate([edge_index[1], loop,
                           jnp.full((pad,), _N, _i32)])
    src16 = src.reshape(_NT, _ECH16, _EC)
    dst16 = dst.reshape(_NT, _ECH16, _EC)
    dst32 = dst.reshape(2 * _NT, _ECH32, _EC)

    npad = _NP2 - _N
    xp = jnp.concatenate([x, jnp.zeros((npad,), _i32)]).reshape(
        2 * _NT, _NCH, _EC)
    idp = jnp.concatenate([loop, jnp.zeros((npad,), _i32)]).reshape(
        2 * _NT, _NCH, _EC)
    bp = jnp.concatenate([batch, jnp.full((npad,), _G, _i32)]).reshape(
        2 * _NT, _NCH, _EC)

    z1 = jnp.zeros((_NR,), _f32)
    z2 = jnp.zeros((_NR, _H), _f32)
    b1r = b1.reshape(1, _D)
    b2r = b2.reshape(1, _D)
    bcr = bc.reshape(1, 2)

    h0, dega, degb = _sc_prep(emb, xp, dst32, z1)
    dega = dega.reshape(1, _NR)
    degb = degb.reshape(1, _NR)
    g1a, g1b = _tc_g1(h0, dega, degb, W1)
    a1a, a1b = _sc_edge(g1a, g1b, src16, dst16, z2)
    g2a, g2b = _tc_mid(a1a, a1b, dega, degb, b1r, W2)
    a2a, a2b = _sc_edge(g2a, g2b, src16, dst16, z2)
    h2 = _tc_h2(a2a, a2b, dega, degb, b2r)
    sumsa, sumsb, cntsa, cntsb = _sc_pool(h2, idp, bp, z1)
    return _tc_out(sumsa, sumsb, cntsa.reshape(1, _GP),
                   cntsb.reshape(1, _GP), Wc, bcr)


# speedup vs baseline: 20.4575x; 1.2977x over previous
import functools

import jax
import jax.numpy as jnp
from jax import lax
from jax.experimental import pallas as pl
from jax.experimental.pallas import tpu as pltpu
from jax.experimental.pallas import tpu_sc as plsc

_N = 50000
_E = 800000
_D = 64
_G = 128
_H = _D // 2

_BLK = 512
_GRID = 98
_NR = _GRID * _BLK

_NT = 16
_NCORE = 2
_EC = 128

_EP = _E + _N
_EPAD = 851968
_ECH16 = _EPAD // (_NT * _EC)
_ECH32 = _EPAD // (2 * _NT * _EC)
_EBLK = 52

_NP2 = 53248
_NCH = _NP2 // (2 * _NT * _EC)
_ZR = _NR // _NT

_GP = 256

_mesh = plsc.VectorSubcoreMesh(core_axis_name="c", subcore_axis_name="s")
_f32 = jnp.float32
_i32 = jnp.int32


def _fill_ones(ref, n):
    for i in range(n // 16):
        ref[pl.ds(i * 16, 16)] = jnp.ones((16,), _f32)


@functools.partial(
    pl.kernel,
    out_type=[
        jax.ShapeDtypeStruct((_NP2, _D), _f32),
        jax.ShapeDtypeStruct((_NR,), _f32),
        jax.ShapeDtypeStruct((_NR,), _f32),
    ],
    mesh=_mesh,
    compiler_params=pltpu.CompilerParams(use_tc_tiling_on_sc=False),
    scratch_types=[
        pltpu.VMEM((_NCH, _EC), _i32),
        pltpu.VMEM((_ECH32, _EC), _i32),
        pltpu.VMEM((_EC, _D), _f32),
        pltpu.VMEM((_EC,), _f32),
        pltpu.VMEM_SHARED((_NR,), _f32),
        pltpu.SemaphoreType.DMA,
    ],
)
def _sc_prep(emb_h, xp_h, dst32_h, z1_h, h0_h, dega_h, degb_h,
             xbuf, dbuf, rbuf, ones_v, deg_s, sem):
    c = lax.axis_index("c")
    s = lax.axis_index("s")
    wid = s * _NCORE + c

    @pl.when(s == 0)
    def _():
        pltpu.sync_copy(z1_h, deg_s)

    _fill_ones(ones_v, _EC)

    pltpu.sync_copy(xp_h.at[wid], xbuf)
    base = wid * _NCH * _EC
    for j in range(_NCH):
        pltpu.async_copy(emb_h.at[xbuf.at[j]], rbuf, sem).wait()
        pltpu.sync_copy(rbuf, h0_h.at[pl.ds(base + j * _EC, _EC)])

    pltpu.sync_copy(dst32_h.at[wid], dbuf)
    plsc.subcore_barrier()

    def deg_body(j, carry):
        pltpu.sync_copy(ones_v, deg_s.at[dbuf.at[j]], add=True)
        return carry

    lax.fori_loop(0, _ECH32, deg_body, 0)
    plsc.subcore_barrier()

    @pl.when(jnp.logical_and(s == 0, c == 0))
    def _():
        pltpu.sync_copy(deg_s, dega_h)

    @pl.when(jnp.logical_and(s == 0, c == 1))
    def _():
        pltpu.sync_copy(deg_s, degb_h)


@functools.partial(
    pl.kernel,
    out_type=[
        jax.ShapeDtypeStruct((_NR, _H), _f32),
        jax.ShapeDtypeStruct((_NR, _H), _f32),
    ],
    mesh=_mesh,
    compiler_params=pltpu.CompilerParams(use_tc_tiling_on_sc=False),
    scratch_types=[
        pltpu.VMEM((_EBLK, _EC), _i32),
        pltpu.VMEM((_EBLK, _EC), _i32),
        pltpu.VMEM((_EC, _H), _f32),
        pltpu.VMEM((_EC, _H), _f32),
        pltpu.VMEM_SHARED((_NR, _H), _f32),
        pltpu.SemaphoreType.DMA,
        pltpu.SemaphoreType.DMA,
    ],
)
def _sc_edge(ga_h, gb_h, src16_h, dst16_h, z2_h, outa_h, outb_h,
             sbuf, dbuf, rb0, rb1, acc_s, sem0, sem1):
    c = lax.axis_index("c")
    s = lax.axis_index("s")

    pltpu.sync_copy(z2_h.at[pl.ds(0, _ZR)], acc_s.at[pl.ds(s * _ZR, _ZR)])
    plsc.subcore_barrier()

    def run(g_h, out_h):
        def start(j, rb, sem):
            pltpu.async_copy(g_h.at[sbuf.at[j]], rb, sem)

        def fin(rb, sem):
            pltpu.make_async_copy(g_h.at[sbuf.at[0]], rb, sem).wait()

        def scat(j, rb):
            pltpu.sync_copy(rb, acc_s.at[dbuf.at[j]], add=True)

        for blk in range(_ECH16 // _EBLK):
            pltpu.sync_copy(src16_h.at[s, pl.ds(blk * _EBLK, _EBLK)], sbuf)
            pltpu.sync_copy(dst16_h.at[s, pl.ds(blk * _EBLK, _EBLK)], dbuf)
            start(0, rb0, sem0)

            def body(m, carry):
                j0 = 2 * m
                start(j0 + 1, rb1, sem1)
                fin(rb0, sem0)
                scat(j0, rb0)
                start(j0 + 2, rb0, sem0)
                fin(rb1, sem1)
                scat(j0 + 1, rb1)
                return carry

            lax.fori_loop(0, _EBLK // 2 - 1, body, 0)
            j0 = _EBLK - 2
            start(j0 + 1, rb1, sem1)
            fin(rb0, sem0)
            scat(j0, rb0)
            fin(rb1, sem1)
            scat(j0 + 1, rb1)
        plsc.subcore_barrier()
        pltpu.sync_copy(acc_s.at[pl.ds(s * _ZR, _ZR)],
                        out_h.at[pl.ds(s * _ZR, _ZR)])

    @pl.when(c == 0)
    def _():
        run(ga_h, outa_h)

    @pl.when(c == 1)
    def _():
        run(gb_h, outb_h)


@functools.partial(
    pl.kernel,
    out_type=[
        jax.ShapeDtypeStruct((_GP, _D), _f32),
        jax.ShapeDtypeStruct((_GP, _D), _f32),
        jax.ShapeDtypeStruct((_GP,), _f32),
        jax.ShapeDtypeStruct((_GP,), _f32),
    ],
    mesh=_mesh,
    compiler_params=pltpu.CompilerParams(use_tc_tiling_on_sc=False),
    scratch_types=[
        pltpu.VMEM((_NCH, _EC), _i32),
        pltpu.VMEM((_NCH, _EC), _i32),
        pltpu.VMEM((_EC, _D), _f32),
        pltpu.VMEM((_EC,), _f32),
        pltpu.VMEM_SHARED((_GP, _D), _f32),
        pltpu.VMEM_SHARED((_GP,), _f32),
        pltpu.SemaphoreType.DMA,
    ],
)
def _sc_pool(h2_h, idp_h, bp_h, z1_h, sumsa_h, sumsb_h, cntsa_h, cntsb_h,
             ibuf, bbuf, rbuf, ones_v, sums_s, cnts_s, sem):
    c = lax.axis_index("c")
    s = lax.axis_index("s")
    wid = s * _NCORE + c
    rows = _GP // _NT

    for r in range(rows):
        for k in range(_D // 16):
            rbuf[r, pl.ds(k * 16, 16)] = jnp.zeros((16,), _f32)
    pltpu.sync_copy(rbuf.at[pl.ds(0, rows)], sums_s.at[pl.ds(s * rows, rows)])
    ones_v[pl.ds(0, 16)] = jnp.zeros((16,), _f32)
    pltpu.sync_copy(ones_v.at[pl.ds(0, rows)],
                    cnts_s.at[pl.ds(s * rows, rows)])
    _fill_ones(ones_v, _EC)
    plsc.subcore_barrier()

    pltpu.sync_copy(idp_h.at[wid], ibuf)
    pltpu.sync_copy(bp_h.at[wid], bbuf)

    def body(j, carry):
        pltpu.async_copy(h2_h.at[ibuf.at[j]], rbuf, sem).wait()
        pltpu.sync_copy(rbuf, sums_s.at[bbuf.at[j]], add=True)
        pltpu.sync_copy(ones_v, cnts_s.at[bbuf.at[j]], add=True)
        return carry

    lax.fori_loop(0, _NCH, body, 0)
    plsc.subcore_barrier()

    @pl.when(jnp.logical_and(s == 0, c == 0))
    def _():
        pltpu.sync_copy(sums_s, sumsa_h)
        pltpu.sync_copy(cnts_s, cntsa_h)

    @pl.when(jnp.logical_and(s == 0, c == 1))
    def _():
        pltpu.sync_copy(sums_s, sumsb_h)
        pltpu.sync_copy(cnts_s, cntsb_h)


def _dinv_of(da_b, db_b):
    deg = da_b[0, :] + db_b[0, :]
    return lax.rsqrt(jnp.maximum(deg, 1.0))


def _tc_g1_body(h0_b, da_b, db_b, w1_b, ga_b, gb_b):
    dinv = _dinv_of(da_b, db_b)
    g = jnp.dot(h0_b[...], w1_b[...],
                preferred_element_type=_f32) * dinv[:, None]
    ga_b[...] = g[:, :_H]
    gb_b[...] = g[:, _H:]


def _tc_mid_body(aa_b, ab_b, da_b, db_b, b1_b, w2_b, ga_b, gb_b):
    dinv = _dinv_of(da_b, db_b)
    agg = jnp.concatenate([aa_b[...], ab_b[...]], axis=1)
    h1 = jnp.maximum(agg * dinv[:, None] + b1_b[...], 0.0)
    g2 = jnp.dot(h1, w2_b[...], preferred_element_type=_f32) * dinv[:, None]
    ga_b[...] = g2[:, :_H]
    gb_b[...] = g2[:, _H:]


def _tc_h2_body(aa_b, ab_b, da_b, db_b, b2_b, h2_b):
    dinv = _dinv_of(da_b, db_b)
    agg = jnp.concatenate([aa_b[...], ab_b[...]], axis=1)
    h2_b[...] = jnp.maximum(agg * dinv[:, None] + b2_b[...], 0.0)


def _tc_out_body(sa_b, sb_b, ca_b, cb_b, wc_b, bc_b, o_b):
    sums = sa_b[...] + sb_b[...]
    cnts = ca_b[0, :] + cb_b[0, :]
    pooled = sums[:_G] / jnp.maximum(cnts[:_G], 1.0)[:, None]
    o_b[...] = jnp.dot(pooled, wc_b[...],
                       preferred_element_type=_f32) + bc_b[...]


def _row_spec(w):
    return pl.BlockSpec((_BLK, w), lambda i: (i, 0))


def _full_spec(shape):
    nd = len(shape)
    return pl.BlockSpec(shape, lambda i, _nd=nd: (0,) * _nd)


_deg_spec = pl.BlockSpec((1, _BLK), lambda i: (0, i))


def _tc_g1(h0, dega, degb, W1):
    return pl.pallas_call(
        _tc_g1_body,
        grid=(_GRID,),
        in_specs=[_row_spec(_D), _deg_spec, _deg_spec,
                  _full_spec((_D, _D))],
        out_specs=[_row_spec(_H), _row_spec(_H)],
        out_shape=[jax.ShapeDtypeStruct((_NR, _H), _f32)] * 2,
    )(h0, dega, degb, W1)


def _tc_mid(aa, ab, dega, degb, b1r, W2):
    return pl.pallas_call(
        _tc_mid_body,
        grid=(_GRID,),
        in_specs=[_row_spec(_H), _row_spec(_H), _deg_spec, _deg_spec,
                  _full_spec((1, _D)), _full_spec((_D, _D))],
        out_specs=[_row_spec(_H), _row_spec(_H)],
        out_shape=[jax.ShapeDtypeStruct((_NR, _H), _f32)] * 2,
    )(aa, ab, dega, degb, b1r, W2)


def _tc_h2(aa, ab, dega, degb, b2r):
    return pl.pallas_call(
        _tc_h2_body,
        grid=(_GRID,),
        in_specs=[_row_spec(_H), _row_spec(_H), _deg_spec, _deg_spec,
                  _full_spec((1, _D))],
        out_specs=_row_spec(_D),
        out_shape=jax.ShapeDtypeStruct((_NR, _D), _f32),
    )(aa, ab, dega, degb, b2r)


def _tc_out(sumsa, sumsb, cntsa, cntsb, Wc, bcr):
    return pl.pallas_call(
        _tc_out_body,
        out_shape=jax.ShapeDtypeStruct((_G, 2), _f32),
    )(sumsa, sumsb, cntsa, cntsb, Wc, bcr)


def kernel(x, edge_index, batch, emb, W1, b1, W2, b2, Wc, bc):
    loop = jnp.arange(_N, dtype=_i32)
    pad = _EPAD - _EP
    src = jnp.concatenate([edge_index[0], loop,
                           jnp.zeros((pad,), _i32)])
    dst = jnp.concatenate([edge_index[1], loop,
                           jnp.full((pad,), _N, _i32)])
    src16 = src.reshape(_NT, _ECH16, _EC)
    dst16 = dst.reshape(_NT, _ECH16, _EC)
    dst32 = dst.reshape(2 * _NT, _ECH32, _EC)

    npad = _NP2 - _N
    xp = jnp.concatenate([x, jnp.zeros((npad,), _i32)]).reshape(
        2 * _NT, _NCH, _EC)
    idp = jnp.concatenate([loop, jnp.zeros((npad,), _i32)]).reshape(
        2 * _NT, _NCH, _EC)
    bp = jnp.concatenate([batch, jnp.full((npad,), _G, _i32)]).reshape(
        2 * _NT, _NCH, _EC)

    z1 = jnp.zeros((_NR,), _f32)
    z2 = jnp.zeros((_NR, _H), _f32)
    b1r = b1.reshape(1, _D)
    b2r = b2.reshape(1, _D)
    bcr = bc.reshape(1, 2)

    h0, dega, degb = _sc_prep(emb, xp, dst32, z1)
    dega = dega.reshape(1, _NR)
    degb = degb.reshape(1, _NR)
    g1a, g1b = _tc_g1(h0, dega, degb, W1)
    a1a, a1b = _sc_edge(g1a, g1b, src16, dst16, z2)
    g2a, g2b = _tc_mid(a1a, a1b, dega, degb, b1r, W2)
    a2a, a2b = _sc_edge(g2a, g2b, src16, dst16, z2)
    h2 = _tc_h2(a2a, a2b, dega, degb, b2r)
    sumsa, sumsb, cntsa, cntsb = _sc_pool(h2, idp, bp, z1)
    return _tc_out(sumsa, sumsb, cntsa.reshape(1, _GP),
                   cntsb.reshape(1, _GP), Wc, bcr)


# speedup vs baseline: 23.7836x; 1.1626x over previous
import functools

import jax
import jax.numpy as jnp
from jax import lax
from jax.experimental import pallas as pl
from jax.experimental.pallas import tpu as pltpu
from jax.experimental.pallas import tpu_sc as plsc

_N = 50000
_E = 800000
_D = 64
_G = 128
_H = _D // 2

_BLK = 2048
_GRID = 25
_NR = _GRID * _BLK

_NT = 16
_NCORE = 2
_EC = 125

_ECH16 = _E // (_NT * _EC)
_ECH32 = _E // (2 * _NT * _EC)
_EBLK = 50

_NP2 = 52000
_NCH = _NP2 // (2 * _NT * _EC)
_ZR = _NR // _NT

_GP = 256

_mesh = plsc.VectorSubcoreMesh(core_axis_name="c", subcore_axis_name="s")
_f32 = jnp.float32
_i32 = jnp.int32


def _fill_ones(ref, n):
    for i in range(n // 16):
        ref[pl.ds(i * 16, 16)] = jnp.ones((16,), _f32)


@functools.partial(
    pl.kernel,
    out_type=[
        jax.ShapeDtypeStruct((_NP2, _D), _f32),
        jax.ShapeDtypeStruct((_NR,), _f32),
        jax.ShapeDtypeStruct((_NR,), _f32),
    ],
    mesh=_mesh,
    compiler_params=pltpu.CompilerParams(use_tc_tiling_on_sc=False),
    scratch_types=[
        pltpu.VMEM((_NCH, _EC), _i32),
        pltpu.VMEM((_ECH32, _EC), _i32),
        pltpu.VMEM((_EC, _D), _f32),
        pltpu.VMEM((128,), _f32),
        pltpu.VMEM_SHARED((_NR,), _f32),
        pltpu.SemaphoreType.DMA,
    ],
)
def _sc_prep(emb_h, xp_h, dst32_h, z1_h, h0_h, dega_h, degb_h,
             xbuf, dbuf, rbuf, ones_v, deg_s, sem):
    c = lax.axis_index("c")
    s = lax.axis_index("s")
    wid = s * _NCORE + c

    @pl.when(s == 0)
    def _():
        pltpu.sync_copy(z1_h, deg_s)

    _fill_ones(ones_v, 128)

    pltpu.sync_copy(xp_h.at[wid], xbuf)
    base = wid * _NCH * _EC
    for j in range(_NCH):
        pltpu.async_copy(emb_h.at[xbuf.at[j]], rbuf, sem).wait()
        pltpu.sync_copy(rbuf, h0_h.at[pl.ds(base + j * _EC, _EC)])

    pltpu.sync_copy(dst32_h.at[wid], dbuf)
    plsc.subcore_barrier()

    def deg_body(j, carry):
        pltpu.sync_copy(ones_v.at[pl.ds(0, _EC)], deg_s.at[dbuf.at[j]],
                        add=True)
        return carry

    lax.fori_loop(0, _ECH32, deg_body, 0)
    plsc.subcore_barrier()

    @pl.when(jnp.logical_and(s == 0, c == 0))
    def _():
        pltpu.sync_copy(deg_s, dega_h)

    @pl.when(jnp.logical_and(s == 0, c == 1))
    def _():
        pltpu.sync_copy(deg_s, degb_h)


@functools.partial(
    pl.kernel,
    out_type=[
        jax.ShapeDtypeStruct((_NR, _H), _f32),
        jax.ShapeDtypeStruct((_NR, _H), _f32),
    ],
    mesh=_mesh,
    compiler_params=pltpu.CompilerParams(use_tc_tiling_on_sc=False),
    scratch_types=[
        pltpu.VMEM((_EBLK, _EC), _i32),
        pltpu.VMEM((_EBLK, _EC), _i32),
        pltpu.VMEM((_EC, _H), _f32),
        pltpu.VMEM((_EC, _H), _f32),
        pltpu.VMEM_SHARED((_NR, _H), _f32),
        pltpu.SemaphoreType.DMA,
        pltpu.SemaphoreType.DMA,
    ],
)
def _sc_edge(ga_h, gb_h, src16_h, dst16_h, z2_h, outa_h, outb_h,
             sbuf, dbuf, rb0, rb1, acc_s, sem0, sem1):
    c = lax.axis_index("c")
    s = lax.axis_index("s")

    pltpu.sync_copy(z2_h.at[pl.ds(0, _ZR)], acc_s.at[pl.ds(s * _ZR, _ZR)])
    plsc.subcore_barrier()

    def run(g_h, out_h):
        def start(j, rb, sem):
            pltpu.async_copy(g_h.at[sbuf.at[j]], rb, sem)

        def fin(rb, sem):
            pltpu.make_async_copy(g_h.at[sbuf.at[0]], rb, sem).wait()

        def scat(j, rb):
            pltpu.sync_copy(rb, acc_s.at[dbuf.at[j]], add=True)

        for blk in range(_ECH16 // _EBLK):
            pltpu.sync_copy(src16_h.at[s, pl.ds(blk * _EBLK, _EBLK)], sbuf)
            pltpu.sync_copy(dst16_h.at[s, pl.ds(blk * _EBLK, _EBLK)], dbuf)
            start(0, rb0, sem0)

            def body(m, carry):
                j0 = 2 * m
                start(j0 + 1, rb1, sem1)
                fin(rb0, sem0)
                scat(j0, rb0)
                start(j0 + 2, rb0, sem0)
                fin(rb1, sem1)
                scat(j0 + 1, rb1)
                return carry

            lax.fori_loop(0, _EBLK // 2 - 1, body, 0)
            j0 = _EBLK - 2
            start(j0 + 1, rb1, sem1)
            fin(rb0, sem0)
            scat(j0, rb0)
            fin(rb1, sem1)
            scat(j0 + 1, rb1)
        plsc.subcore_barrier()
        pltpu.sync_copy(acc_s.at[pl.ds(s * _ZR, _ZR)],
                        out_h.at[pl.ds(s * _ZR, _ZR)])

    @pl.when(c == 0)
    def _():
        run(ga_h, outa_h)

    @pl.when(c == 1)
    def _():
        run(gb_h, outb_h)


@functools.partial(
    pl.kernel,
    out_type=[
        jax.ShapeDtypeStruct((_GP, _D), _f32),
        jax.ShapeDtypeStruct((_GP, _D), _f32),
        jax.ShapeDtypeStruct((_GP,), _f32),
        jax.ShapeDtypeStruct((_GP,), _f32),
    ],
    mesh=_mesh,
    compiler_params=pltpu.CompilerParams(use_tc_tiling_on_sc=False),
    scratch_types=[
        pltpu.VMEM((_NCH, _EC), _i32),
        pltpu.VMEM((_NCH, _EC), _i32),
        pltpu.VMEM((_EC, _D), _f32),
        pltpu.VMEM((128,), _f32),
        pltpu.VMEM_SHARED((_GP, _D), _f32),
        pltpu.VMEM_SHARED((_GP,), _f32),
        pltpu.SemaphoreType.DMA,
    ],
)
def _sc_pool(h2_h, idp_h, bp_h, z1_h, sumsa_h, sumsb_h, cntsa_h, cntsb_h,
             ibuf, bbuf, rbuf, ones_v, sums_s, cnts_s, sem):
    c = lax.axis_index("c")
    s = lax.axis_index("s")
    wid = s * _NCORE + c
    rows = _GP // _NT

    for r in range(rows):
        for k in range(_D // 16):
            rbuf[r, pl.ds(k * 16, 16)] = jnp.zeros((16,), _f32)
    pltpu.sync_copy(rbuf.at[pl.ds(0, rows)], sums_s.at[pl.ds(s * rows, rows)])
    ones_v[pl.ds(0, 16)] = jnp.zeros((16,), _f32)
    pltpu.sync_copy(ones_v.at[pl.ds(0, rows)],
                    cnts_s.at[pl.ds(s * rows, rows)])
    _fill_ones(ones_v, 128)
    plsc.subcore_barrier()

    pltpu.sync_copy(idp_h.at[wid], ibuf)
    pltpu.sync_copy(bp_h.at[wid], bbuf)

    def body(j, carry):
        pltpu.async_copy(h2_h.at[ibuf.at[j]], rbuf, sem).wait()
        pltpu.sync_copy(rbuf, sums_s.at[bbuf.at[j]], add=True)
        pltpu.sync_copy(ones_v.at[pl.ds(0, _EC)], cnts_s.at[bbuf.at[j]],
                        add=True)
        return carry

    lax.fori_loop(0, _NCH, body, 0)
    plsc.subcore_barrier()

    @pl.when(jnp.logical_and(s == 0, c == 0))
    def _():
        pltpu.sync_copy(sums_s, sumsa_h)
        pltpu.sync_copy(cnts_s, cntsa_h)

    @pl.when(jnp.logical_and(s == 0, c == 1))
    def _():
        pltpu.sync_copy(sums_s, sumsb_h)
        pltpu.sync_copy(cnts_s, cntsb_h)


def _dinv_of(da_b, db_b):
    deg = da_b[0, :] + db_b[0, :] + 1.0
    return lax.rsqrt(deg)


def _tc_g1_body(h0_b, da_b, db_b, w1_b, ga_b, gb_b):
    dinv = _dinv_of(da_b, db_b)
    g = jnp.dot(h0_b[...], w1_b[...],
                preferred_element_type=_f32) * dinv[:, None]
    ga_b[...] = g[:, :_H]
    gb_b[...] = g[:, _H:]


def _tc_mid_body(aa_b, ab_b, pa_b, pb_b, da_b, db_b, b1_b, w2_b,
                 ga_b, gb_b):
    dinv = _dinv_of(da_b, db_b)
    agg = jnp.concatenate([aa_b[...] + pa_b[...], ab_b[...] + pb_b[...]],
                          axis=1)
    h1 = jnp.maximum(agg * dinv[:, None] + b1_b[...], 0.0)
    g2 = jnp.dot(h1, w2_b[...], preferred_element_type=_f32) * dinv[:, None]
    ga_b[...] = g2[:, :_H]
    gb_b[...] = g2[:, _H:]


def _tc_h2_body(aa_b, ab_b, pa_b, pb_b, da_b, db_b, b2_b, h2_b):
    dinv = _dinv_of(da_b, db_b)
    agg = jnp.concatenate([aa_b[...] + pa_b[...], ab_b[...] + pb_b[...]],
                          axis=1)
    h2_b[...] = jnp.maximum(agg * dinv[:, None] + b2_b[...], 0.0)


def _tc_out_body(sa_b, sb_b, ca_b, cb_b, wc_b, bc_b, o_b):
    sums = sa_b[...] + sb_b[...]
    cnts = ca_b[0, :] + cb_b[0, :]
    pooled = sums[:_G] / jnp.maximum(cnts[:_G], 1.0)[:, None]
    o_b[...] = jnp.dot(pooled, wc_b[...],
                       preferred_element_type=_f32) + bc_b[...]


def _row_spec(w):
    return pl.BlockSpec((_BLK, w), lambda i: (i, 0))


def _full_spec(shape):
    nd = len(shape)
    return pl.BlockSpec(shape, lambda i, _nd=nd: (0,) * _nd)


_deg_spec = pl.BlockSpec((1, _BLK), lambda i: (0, i))


def _tc_g1(h0, dega, degb, W1):
    return pl.pallas_call(
        _tc_g1_body,
        grid=(_GRID,),
        in_specs=[_row_spec(_D), _deg_spec, _deg_spec,
                  _full_spec((_D, _D))],
        out_specs=[_row_spec(_H), _row_spec(_H)],
        out_shape=[jax.ShapeDtypeStruct((_NR, _H), _f32)] * 2,
    )(h0, dega, degb, W1)


def _tc_mid(aa, ab, pa, pb, dega, degb, b1r, W2):
    return pl.pallas_call(
        _tc_mid_body,
        grid=(_GRID,),
        in_specs=[_row_spec(_H), _row_spec(_H), _row_spec(_H), _row_spec(_H),
                  _deg_spec, _deg_spec,
                  _full_spec((1, _D)), _full_spec((_D, _D))],
        out_specs=[_row_spec(_H), _row_spec(_H)],
        out_shape=[jax.ShapeDtypeStruct((_NR, _H), _f32)] * 2,
    )(aa, ab, pa, pb, dega, degb, b1r, W2)


def _tc_h2(aa, ab, pa, pb, dega, degb, b2r):
    return pl.pallas_call(
        _tc_h2_body,
        grid=(_GRID,),
        in_specs=[_row_spec(_H), _row_spec(_H), _row_spec(_H), _row_spec(_H),
                  _deg_spec, _deg_spec, _full_spec((1, _D))],
        out_specs=_row_spec(_D),
        out_shape=jax.ShapeDtypeStruct((_NR, _D), _f32),
    )(aa, ab, pa, pb, dega, degb, b2r)


def _tc_out(sumsa, sumsb, cntsa, cntsb, Wc, bcr):
    return pl.pallas_call(
        _tc_out_body,
        out_shape=jax.ShapeDtypeStruct((_G, 2), _f32),
    )(sumsa, sumsb, cntsa, cntsb, Wc, bcr)


def kernel(x, edge_index, batch, emb, W1, b1, W2, b2, Wc, bc):
    loop = jnp.arange(_N, dtype=_i32)
    src16 = edge_index[0].reshape(_NT, _ECH16, _EC)
    dst16 = edge_index[1].reshape(_NT, _ECH16, _EC)
    dst32 = edge_index[1].reshape(2 * _NT, _ECH32, _EC)

    npad = _NP2 - _N
    xp = jnp.concatenate([x, jnp.zeros((npad,), _i32)]).reshape(
        2 * _NT, _NCH, _EC)
    idp = jnp.concatenate([loop, jnp.zeros((npad,), _i32)]).reshape(
        2 * _NT, _NCH, _EC)
    bp = jnp.concatenate([batch, jnp.full((npad,), _G, _i32)]).reshape(
        2 * _NT, _NCH, _EC)

    z1 = jnp.zeros((_NR,), _f32)
    z2 = jnp.zeros((_NR, _H), _f32)
    b1r = b1.reshape(1, _D)
    b2r = b2.reshape(1, _D)
    bcr = bc.reshape(1, 2)

    h0, dega, degb = _sc_prep(emb, xp, dst32, z1)
    dega = dega.reshape(1, _NR)
    degb = degb.reshape(1, _NR)
    g1a, g1b = _tc_g1(h0, dega, degb, W1)
    a1a, a1b = _sc_edge(g1a, g1b, src16, dst16, z2)
    g2a, g2b = _tc_mid(a1a, a1b, g1a, g1b, dega, degb, b1r, W2)
    a2a, a2b = _sc_edge(g2a, g2b, src16, dst16, z2)
    h2 = _tc_h2(a2a, a2b, g2a, g2b, dega, degb, b2r)
    sumsa, sumsb, cntsa, cntsb = _sc_pool(h2, idp, bp, z1)
    return _tc_out(sumsa, sumsb, cntsa.reshape(1, _GP),
                   cntsb.reshape(1, _GP), Wc, bcr)


# speedup vs baseline: 27.4161x; 1.1527x over previous
import functools

import jax
import jax.numpy as jnp
from jax import lax
from jax.experimental import pallas as pl
from jax.experimental.pallas import tpu as pltpu
from jax.experimental.pallas import tpu_sc as plsc

_N = 50000
_E = 800000
_D = 64
_G = 128
_H = _D // 2

_BLK = 2048
_GRID = 25
_NR = _GRID * _BLK

_NT = 16
_NCORE = 2
_EC = 125

_ECH16 = _E // (_NT * _EC)
_ECH32 = _E // (2 * _NT * _EC)
_EBLK = 40

_NP2 = 52000
_NCH = _NP2 // (2 * _NT * _EC)
_ZR = _NR // _NT

_GP = 256

_mesh = plsc.VectorSubcoreMesh(core_axis_name="c", subcore_axis_name="s")
_f32 = jnp.float32
_i32 = jnp.int32


def _fill_ones(ref, n):
    for i in range(n // 16):
        ref[pl.ds(i * 16, 16)] = jnp.ones((16,), _f32)


@functools.partial(
    pl.kernel,
    out_type=[
        jax.ShapeDtypeStruct((_NP2, _D), _f32),
        jax.ShapeDtypeStruct((_NR,), _f32),
        jax.ShapeDtypeStruct((_NR,), _f32),
    ],
    mesh=_mesh,
    compiler_params=pltpu.CompilerParams(use_tc_tiling_on_sc=False),
    scratch_types=[
        pltpu.VMEM((_NCH, _EC), _i32),
        pltpu.VMEM((_ECH32, _EC), _i32),
        pltpu.VMEM((_EC, _D), _f32),
        pltpu.VMEM((128,), _f32),
        pltpu.VMEM_SHARED((_NR,), _f32),
        pltpu.SemaphoreType.DMA,
    ],
)
def _sc_prep(emb_h, xp_h, dst32_h, z1_h, h0_h, dega_h, degb_h,
             xbuf, dbuf, rbuf, ones_v, deg_s, sem):
    c = lax.axis_index("c")
    s = lax.axis_index("s")
    wid = s * _NCORE + c

    @pl.when(s == 0)
    def _():
        pltpu.sync_copy(z1_h, deg_s)

    _fill_ones(ones_v, 128)

    pltpu.sync_copy(xp_h.at[wid], xbuf)
    base = wid * _NCH * _EC
    for j in range(_NCH):
        pltpu.async_copy(emb_h.at[xbuf.at[j]], rbuf, sem).wait()
        pltpu.sync_copy(rbuf, h0_h.at[pl.ds(base + j * _EC, _EC)])

    pltpu.sync_copy(dst32_h.at[wid], dbuf)
    plsc.subcore_barrier()

    def deg_body(j, carry):
        pltpu.sync_copy(ones_v.at[pl.ds(0, _EC)], deg_s.at[dbuf.at[j]],
                        add=True)
        return carry

    lax.fori_loop(0, _ECH32, deg_body, 0)
    plsc.subcore_barrier()

    @pl.when(jnp.logical_and(s == 0, c == 0))
    def _():
        pltpu.sync_copy(deg_s, dega_h)

    @pl.when(jnp.logical_and(s == 0, c == 1))
    def _():
        pltpu.sync_copy(deg_s, degb_h)


@functools.partial(
    pl.kernel,
    out_type=[
        jax.ShapeDtypeStruct((_NR, _H), _f32),
        jax.ShapeDtypeStruct((_NR, _H), _f32),
    ],
    mesh=_mesh,
    compiler_params=pltpu.CompilerParams(use_tc_tiling_on_sc=False),
    scratch_types=[
        pltpu.VMEM((_EBLK, _EC), _i32),
        pltpu.VMEM((_EBLK, _EC), _i32),
        [pltpu.VMEM((_EC, _H), _f32)] * 4,
        pltpu.VMEM_SHARED((_NR, _H), _f32),
        [pltpu.SemaphoreType.DMA] * 4,
        [pltpu.SemaphoreType.DMA] * 4,
    ],
)
def _sc_edge(ga_h, gb_h, src16_h, dst16_h, z2_h, outa_h, outb_h,
             sbuf, dbuf, rbs, acc_s, gsems, ssems):
    c = lax.axis_index("c")
    s = lax.axis_index("s")

    pltpu.sync_copy(z2_h.at[pl.ds(0, _ZR)], acc_s.at[pl.ds(s * _ZR, _ZR)])
    plsc.subcore_barrier()

    def run(g_h, out_h):
        def start_g(j, b):
            pltpu.async_copy(g_h.at[sbuf.at[j]], rbs[b], gsems[b])

        def wait_g(b):
            pltpu.make_async_copy(g_h.at[sbuf.at[0]], rbs[b],
                                  gsems[b]).wait()

        def start_s(j, b):
            pltpu.async_copy(rbs[b], acc_s.at[dbuf.at[j]], ssems[b],
                             add=True)

        def wait_s(b):
            pltpu.make_async_copy(rbs[b], acc_s.at[dbuf.at[0]],
                                  ssems[b]).wait()

        for blk in range(_ECH16 // _EBLK):
            pltpu.sync_copy(src16_h.at[s, pl.ds(blk * _EBLK, _EBLK)], sbuf)
            pltpu.sync_copy(dst16_h.at[s, pl.ds(blk * _EBLK, _EBLK)], dbuf)
            start_g(0, 0)
            start_g(1, 1)
            start_g(2, 2)
            start_g(3, 3)
            wait_g(0)
            start_s(0, 0)
            wait_g(1)
            start_s(1, 1)

            def body(m, carry):
                k = 2 + 4 * m
                for u in range(4):
                    j = k + u
                    bg = u
                    bs = (2 + u) % 4
                    wait_s(bg)
                    start_g(j + 2, bg)
                    wait_g(bs)
                    start_s(j, bs)
                return carry

            lax.fori_loop(0, (_EBLK - 4) // 4, body, 0)
            wait_g(2)
            start_s(_EBLK - 2, 2)
            wait_g(3)
            start_s(_EBLK - 1, 3)
            for b in range(4):
                wait_s(b)
        plsc.subcore_barrier()
        pltpu.sync_copy(acc_s.at[pl.ds(s * _ZR, _ZR)],
                        out_h.at[pl.ds(s * _ZR, _ZR)])

    @pl.when(c == 0)
    def _():
        run(ga_h, outa_h)

    @pl.when(c == 1)
    def _():
        run(gb_h, outb_h)


@functools.partial(
    pl.kernel,
    out_type=[
        jax.ShapeDtypeStruct((_GP, _D), _f32),
        jax.ShapeDtypeStruct((_GP, _D), _f32),
        jax.ShapeDtypeStruct((_GP,), _f32),
        jax.ShapeDtypeStruct((_GP,), _f32),
    ],
    mesh=_mesh,
    compiler_params=pltpu.CompilerParams(use_tc_tiling_on_sc=False),
    scratch_types=[
        pltpu.VMEM((_NCH, _EC), _i32),
        pltpu.VMEM((_NCH, _EC), _i32),
        pltpu.VMEM((_EC, _D), _f32),
        pltpu.VMEM((128,), _f32),
        pltpu.VMEM_SHARED((_GP, _D), _f32),
        pltpu.VMEM_SHARED((_GP,), _f32),
        pltpu.SemaphoreType.DMA,
    ],
)
def _sc_pool(h2_h, idp_h, bp_h, z1_h, sumsa_h, sumsb_h, cntsa_h, cntsb_h,
             ibuf, bbuf, rbuf, ones_v, sums_s, cnts_s, sem):
    c = lax.axis_index("c")
    s = lax.axis_index("s")
    wid = s * _NCORE + c
    rows = _GP // _NT

    for r in range(rows):
        for k in range(_D // 16):
            rbuf[r, pl.ds(k * 16, 16)] = jnp.zeros((16,), _f32)
    pltpu.sync_copy(rbuf.at[pl.ds(0, rows)], sums_s.at[pl.ds(s * rows, rows)])
    ones_v[pl.ds(0, 16)] = jnp.zeros((16,), _f32)
    pltpu.sync_copy(ones_v.at[pl.ds(0, rows)],
                    cnts_s.at[pl.ds(s * rows, rows)])
    _fill_ones(ones_v, 128)
    plsc.subcore_barrier()

    pltpu.sync_copy(idp_h.at[wid], ibuf)
    pltpu.sync_copy(bp_h.at[wid], bbuf)

    def body(j, carry):
        pltpu.async_copy(h2_h.at[ibuf.at[j]], rbuf, sem).wait()
        pltpu.sync_copy(rbuf, sums_s.at[bbuf.at[j]], add=True)
        pltpu.sync_copy(ones_v.at[pl.ds(0, _EC)], cnts_s.at[bbuf.at[j]],
                        add=True)
        return carry

    lax.fori_loop(0, _NCH, body, 0)
    plsc.subcore_barrier()

    @pl.when(jnp.logical_and(s == 0, c == 0))
    def _():
        pltpu.sync_copy(sums_s, sumsa_h)
        pltpu.sync_copy(cnts_s, cntsa_h)

    @pl.when(jnp.logical_and(s == 0, c == 1))
    def _():
        pltpu.sync_copy(sums_s, sumsb_h)
        pltpu.sync_copy(cnts_s, cntsb_h)


def _dinv_of(da_b, db_b):
    deg = da_b[0, :] + db_b[0, :] + 1.0
    return lax.rsqrt(deg)


def _tc_g1_body(h0_b, da_b, db_b, w1_b, ga_b, gb_b):
    dinv = _dinv_of(da_b, db_b)
    g = jnp.dot(h0_b[...], w1_b[...],
                preferred_element_type=_f32) * dinv[:, None]
    ga_b[...] = g[:, :_H]
    gb_b[...] = g[:, _H:]


def _tc_mid_body(aa_b, ab_b, pa_b, pb_b, da_b, db_b, b1_b, w2_b,
                 ga_b, gb_b):
    dinv = _dinv_of(da_b, db_b)
    agg = jnp.concatenate([aa_b[...] + pa_b[...], ab_b[...] + pb_b[...]],
                          axis=1)
    h1 = jnp.maximum(agg * dinv[:, None] + b1_b[...], 0.0)
    g2 = jnp.dot(h1, w2_b[...], preferred_element_type=_f32) * dinv[:, None]
    ga_b[...] = g2[:, :_H]
    gb_b[...] = g2[:, _H:]


def _tc_h2_body(aa_b, ab_b, pa_b, pb_b, da_b, db_b, b2_b, h2_b):
    dinv = _dinv_of(da_b, db_b)
    agg = jnp.concatenate([aa_b[...] + pa_b[...], ab_b[...] + pb_b[...]],
                          axis=1)
    h2_b[...] = jnp.maximum(agg * dinv[:, None] + b2_b[...], 0.0)


def _tc_out_body(sa_b, sb_b, ca_b, cb_b, wc_b, bc_b, o_b):
    sums = sa_b[...] + sb_b[...]
    cnts = ca_b[0, :] + cb_b[0, :]
    pooled = sums[:_G] / jnp.maximum(cnts[:_G], 1.0)[:, None]
    o_b[...] = jnp.dot(pooled, wc_b[...],
                       preferred_element_type=_f32) + bc_b[...]


def _row_spec(w):
    return pl.BlockSpec((_BLK, w), lambda i: (i, 0))


def _full_spec(shape):
    nd = len(shape)
    return pl.BlockSpec(shape, lambda i, _nd=nd: (0,) * _nd)


_deg_spec = pl.BlockSpec((1, _BLK), lambda i: (0, i))


def _tc_g1(h0, dega, degb, W1):
    return pl.pallas_call(
        _tc_g1_body,
        grid=(_GRID,),
        in_specs=[_row_spec(_D), _deg_spec, _deg_spec,
                  _full_spec((_D, _D))],
        out_specs=[_row_spec(_H), _row_spec(_H)],
        out_shape=[jax.ShapeDtypeStruct((_NR, _H), _f32)] * 2,
    )(h0, dega, degb, W1)


def _tc_mid(aa, ab, pa, pb, dega, degb, b1r, W2):
    return pl.pallas_call(
        _tc_mid_body,
        grid=(_GRID,),
        in_specs=[_row_spec(_H), _row_spec(_H), _row_spec(_H), _row_spec(_H),
                  _deg_spec, _deg_spec,
                  _full_spec((1, _D)), _full_spec((_D, _D))],
        out_specs=[_row_spec(_H), _row_spec(_H)],
        out_shape=[jax.ShapeDtypeStruct((_NR, _H), _f32)] * 2,
    )(aa, ab, pa, pb, dega, degb, b1r, W2)


def _tc_h2(aa, ab, pa, pb, dega, degb, b2r):
    return pl.pallas_call(
        _tc_h2_body,
        grid=(_GRID,),
        in_specs=[_row_spec(_H), _row_spec(_H), _row_spec(_H), _row_spec(_H),
                  _deg_spec, _deg_spec, _full_spec((1, _D))],
        out_specs=_row_spec(_D),
        out_shape=jax.ShapeDtypeStruct((_NR, _D), _f32),
    )(aa, ab, pa, pb, dega, degb, b2r)


def _tc_out(sumsa, sumsb, cntsa, cntsb, Wc, bcr):
    return pl.pallas_call(
        _tc_out_body,
        out_shape=jax.ShapeDtypeStruct((_G, 2), _f32),
    )(sumsa, sumsb, cntsa, cntsb, Wc, bcr)


def kernel(x, edge_index, batch, emb, W1, b1, W2, b2, Wc, bc):
    loop = jnp.arange(_N, dtype=_i32)
    src16 = edge_index[0].reshape(_NT, _ECH16, _EC)
    dst16 = edge_index[1].reshape(_NT, _ECH16, _EC)
    dst32 = edge_index[1].reshape(2 * _NT, _ECH32, _EC)

    npad = _NP2 - _N
    xp = jnp.concatenate([x, jnp.zeros((npad,), _i32)]).reshape(
        2 * _NT, _NCH, _EC)
    idp = jnp.concatenate([loop, jnp.zeros((npad,), _i32)]).reshape(
        2 * _NT, _NCH, _EC)
    bp = jnp.concatenate([batch, jnp.full((npad,), _G, _i32)]).reshape(
        2 * _NT, _NCH, _EC)

    z1 = jnp.zeros((_NR,), _f32)
    z2 = jnp.zeros((_NR, _H), _f32)
    b1r = b1.reshape(1, _D)
    b2r = b2.reshape(1, _D)
    bcr = bc.reshape(1, 2)

    h0, dega, degb = _sc_prep(emb, xp, dst32, z1)
    dega = dega.reshape(1, _NR)
    degb = degb.reshape(1, _NR)
    g1a, g1b = _tc_g1(h0, dega, degb, W1)
    a1a, a1b = _sc_edge(g1a, g1b, src16, dst16, z2)
    g2a, g2b = _tc_mid(a1a, a1b, g1a, g1b, dega, degb, b1r, W2)
    a2a, a2b = _sc_edge(g2a, g2b, src16, dst16, z2)
    h2 = _tc_h2(a2a, a2b, g2a, g2b, dega, degb, b2r)
    sumsa, sumsb, cntsa, cntsb = _sc_pool(h2, idp, bp, z1)
    return _tc_out(sumsa, sumsb, cntsa.reshape(1, _GP),
                   cntsb.reshape(1, _GP), Wc, bcr)


# speedup vs baseline: 28.0957x; 1.0248x over previous
import functools

import jax
import jax.numpy as jnp
from jax import lax
from jax.experimental import pallas as pl
from jax.experimental.pallas import tpu as pltpu
from jax.experimental.pallas import tpu_sc as plsc

_N = 50000
_E = 800000
_D = 64
_G = 128
_H = _D // 2

_BLK = 2048
_GRID = 25
_NR = _GRID * _BLK

_NT = 16
_NCORE = 2
_EC = 125

_ECH16 = _E // (_NT * _EC)
_ECH32 = _E // (2 * _NT * _EC)
_EBLK = 40

_NP2 = 52000
_NCH = _NP2 // (2 * _NT * _EC)
_ZR = _NR // _NT

_GP = 256

_mesh = plsc.VectorSubcoreMesh(core_axis_name="c", subcore_axis_name="s")
_f32 = jnp.float32
_i32 = jnp.int32


def _fill_ones(ref, n):
    for i in range(n // 16):
        ref[pl.ds(i * 16, 16)] = jnp.ones((16,), _f32)


@functools.partial(
    pl.kernel,
    out_type=[
        jax.ShapeDtypeStruct((_NP2, _D), _f32),
        jax.ShapeDtypeStruct((_NR,), _f32),
        jax.ShapeDtypeStruct((_NR,), _f32),
    ],
    mesh=_mesh,
    compiler_params=pltpu.CompilerParams(use_tc_tiling_on_sc=False),
    scratch_types=[
        pltpu.VMEM((_NCH, _EC), _i32),
        pltpu.VMEM((_ECH32, _EC), _i32),
        [pltpu.VMEM((_EC, _D), _f32)] * 2,
        pltpu.VMEM((128,), _f32),
        pltpu.VMEM_SHARED((_NR,), _f32),
        [pltpu.SemaphoreType.DMA] * 2,
        [pltpu.SemaphoreType.DMA] * 2,
    ],
)
def _sc_prep(emb_h, xp_h, dst32_h, z1_h, h0_h, dega_h, degb_h,
             xbuf, dbuf, rbufs, ones_v, deg_s, gsems, dsems):
    c = lax.axis_index("c")
    s = lax.axis_index("s")
    wid = s * _NCORE + c

    @pl.when(s == 0)
    def _():
        pltpu.sync_copy(z1_h, deg_s)

    _fill_ones(ones_v, 128)

    pltpu.sync_copy(xp_h.at[wid], xbuf)
    base = wid * _NCH * _EC
    pltpu.async_copy(emb_h.at[xbuf.at[0]], rbufs[0], gsems[0])
    for j in range(_NCH):
        b = j % 2
        if j + 1 < _NCH:
            pltpu.async_copy(emb_h.at[xbuf.at[j + 1]], rbufs[1 - b],
                             gsems[1 - b])
        pltpu.make_async_copy(emb_h.at[xbuf.at[0]], rbufs[b],
                              gsems[b]).wait()
        pltpu.sync_copy(rbufs[b], h0_h.at[pl.ds(base + j * _EC, _EC)])

    pltpu.sync_copy(dst32_h.at[wid], dbuf)
    plsc.subcore_barrier()

    def deg_start(j, k):
        pltpu.async_copy(ones_v.at[pl.ds(0, _EC)], deg_s.at[dbuf.at[j]],
                         dsems[k], add=True)

    def deg_wait(k):
        pltpu.make_async_copy(ones_v.at[pl.ds(0, _EC)],
                              deg_s.at[dbuf.at[0]], dsems[k]).wait()

    deg_start(0, 0)
    deg_start(1, 1)

    def deg_body(m, carry):
        j = 2 + 2 * m
        deg_wait(0)
        deg_start(j, 0)
        deg_wait(1)
        deg_start(j + 1, 1)
        return carry

    lax.fori_loop(0, (_ECH32 - 2) // 2, deg_body, 0)
    deg_wait(0)
    deg_wait(1)
    plsc.subcore_barrier()

    @pl.when(jnp.logical_and(s == 0, c == 0))
    def _():
        pltpu.sync_copy(deg_s, dega_h)

    @pl.when(jnp.logical_and(s == 0, c == 1))
    def _():
        pltpu.sync_copy(deg_s, degb_h)


@functools.partial(
    pl.kernel,
    out_type=[
        jax.ShapeDtypeStruct((_NR, _H), _f32),
        jax.ShapeDtypeStruct((_NR, _H), _f32),
    ],
    mesh=_mesh,
    compiler_params=pltpu.CompilerParams(use_tc_tiling_on_sc=False),
    scratch_types=[
        pltpu.VMEM((_EBLK, _EC), _i32),
        pltpu.VMEM((_EBLK, _EC), _i32),
        [pltpu.VMEM((_EC, _H), _f32)] * 4,
        pltpu.VMEM_SHARED((_NR, _H), _f32),
        [pltpu.SemaphoreType.DMA] * 4,
        [pltpu.SemaphoreType.DMA] * 4,
    ],
)
def _sc_edge(ga_h, gb_h, src16_h, dst16_h, z2_h, outa_h, outb_h,
             sbuf, dbuf, rbs, acc_s, gsems, ssems):
    c = lax.axis_index("c")
    s = lax.axis_index("s")

    pltpu.sync_copy(z2_h.at[pl.ds(0, _ZR)], acc_s.at[pl.ds(s * _ZR, _ZR)])
    plsc.subcore_barrier()

    def run(g_h, out_h):
        def start_g(j, b):
            pltpu.async_copy(g_h.at[sbuf.at[j]], rbs[b], gsems[b])

        def wait_g(b):
            pltpu.make_async_copy(g_h.at[sbuf.at[0]], rbs[b],
                                  gsems[b]).wait()

        def start_s(j, b):
            pltpu.async_copy(rbs[b], acc_s.at[dbuf.at[j]], ssems[b],
                             add=True)

        def wait_s(b):
            pltpu.make_async_copy(rbs[b], acc_s.at[dbuf.at[0]],
                                  ssems[b]).wait()

        for blk in range(_ECH16 // _EBLK):
            pltpu.sync_copy(src16_h.at[s, pl.ds(blk * _EBLK, _EBLK)], sbuf)
            pltpu.sync_copy(dst16_h.at[s, pl.ds(blk * _EBLK, _EBLK)], dbuf)
            start_g(0, 0)
            start_g(1, 1)
            start_g(2, 2)
            start_g(3, 3)
            wait_g(0)
            start_s(0, 0)
            wait_g(1)
            start_s(1, 1)

            def body(m, carry):
                k = 2 + 4 * m
                for u in range(4):
                    j = k + u
                    bg = u
                    bs = (2 + u) % 4
                    wait_s(bg)
                    start_g(j + 2, bg)
                    wait_g(bs)
                    start_s(j, bs)
                return carry

            lax.fori_loop(0, (_EBLK - 4) // 4, body, 0)
            wait_g(2)
            start_s(_EBLK - 2, 2)
            wait_g(3)
            start_s(_EBLK - 1, 3)
            for b in range(4):
                wait_s(b)
        plsc.subcore_barrier()
        pltpu.sync_copy(acc_s.at[pl.ds(s * _ZR, _ZR)],
                        out_h.at[pl.ds(s * _ZR, _ZR)])

    @pl.when(c == 0)
    def _():
        run(ga_h, outa_h)

    @pl.when(c == 1)
    def _():
        run(gb_h, outb_h)


@functools.partial(
    pl.kernel,
    out_type=[
        jax.ShapeDtypeStruct((_GP, _D), _f32),
        jax.ShapeDtypeStruct((_GP, _D), _f32),
        jax.ShapeDtypeStruct((_GP,), _f32),
        jax.ShapeDtypeStruct((_GP,), _f32),
    ],
    mesh=_mesh,
    compiler_params=pltpu.CompilerParams(use_tc_tiling_on_sc=False),
    scratch_types=[
        pltpu.VMEM((_NCH, _EC), _i32),
        pltpu.VMEM((_NCH, _EC), _i32),
        [pltpu.VMEM((_EC, _D), _f32)] * 2,
        pltpu.VMEM((128,), _f32),
        pltpu.VMEM_SHARED((_GP, _D), _f32),
        pltpu.VMEM_SHARED((_GP,), _f32),
        [pltpu.SemaphoreType.DMA] * 2,
    ],
)
def _sc_pool(h2_h, idp_h, bp_h, z1_h, sumsa_h, sumsb_h, cntsa_h, cntsb_h,
             ibuf, bbuf, rbufs, ones_v, sums_s, cnts_s, sems):
    c = lax.axis_index("c")
    s = lax.axis_index("s")
    wid = s * _NCORE + c
    rows = _GP // _NT

    rbuf = rbufs[0]
    for r in range(rows):
        for k in range(_D // 16):
            rbuf[r, pl.ds(k * 16, 16)] = jnp.zeros((16,), _f32)
    pltpu.sync_copy(rbuf.at[pl.ds(0, rows)], sums_s.at[pl.ds(s * rows, rows)])
    ones_v[pl.ds(0, 16)] = jnp.zeros((16,), _f32)
    pltpu.sync_copy(ones_v.at[pl.ds(0, rows)],
                    cnts_s.at[pl.ds(s * rows, rows)])
    _fill_ones(ones_v, 128)
    plsc.subcore_barrier()

    pltpu.sync_copy(idp_h.at[wid], ibuf)
    pltpu.sync_copy(bp_h.at[wid], bbuf)
    pltpu.async_copy(h2_h.at[ibuf.at[0]], rbufs[0], sems[0])
    for j in range(_NCH):
        b = j % 2
        if j + 1 < _NCH:
            pltpu.async_copy(h2_h.at[ibuf.at[j + 1]], rbufs[1 - b],
                             sems[1 - b])
        pltpu.make_async_copy(h2_h.at[ibuf.at[0]], rbufs[b],
                              sems[b]).wait()
        pltpu.sync_copy(rbufs[b], sums_s.at[bbuf.at[j]], add=True)
        pltpu.sync_copy(ones_v.at[pl.ds(0, _EC)], cnts_s.at[bbuf.at[j]],
                        add=True)
    plsc.subcore_barrier()

    @pl.when(jnp.logical_and(s == 0, c == 0))
    def _():
        pltpu.sync_copy(sums_s, sumsa_h)
        pltpu.sync_copy(cnts_s, cntsa_h)

    @pl.when(jnp.logical_and(s == 0, c == 1))
    def _():
        pltpu.sync_copy(sums_s, sumsb_h)
        pltpu.sync_copy(cnts_s, cntsb_h)


def _dinv_of(da_b, db_b):
    deg = da_b[0, :] + db_b[0, :] + 1.0
    return lax.rsqrt(deg)


def _tc_g1_body(h0_b, da_b, db_b, w1_b, ga_b, gb_b):
    dinv = _dinv_of(da_b, db_b)
    g = jnp.dot(h0_b[...], w1_b[...],
                preferred_element_type=_f32) * dinv[:, None]
    ga_b[...] = g[:, :_H]
    gb_b[...] = g[:, _H:]


def _tc_mid_body(aa_b, ab_b, pa_b, pb_b, da_b, db_b, b1_b, w2_b,
                 ga_b, gb_b):
    dinv = _dinv_of(da_b, db_b)
    agg = jnp.concatenate([aa_b[...] + pa_b[...], ab_b[...] + pb_b[...]],
                          axis=1)
    h1 = jnp.maximum(agg * dinv[:, None] + b1_b[...], 0.0)
    g2 = jnp.dot(h1, w2_b[...], preferred_element_type=_f32) * dinv[:, None]
    ga_b[...] = g2[:, :_H]
    gb_b[...] = g2[:, _H:]


def _tc_h2_body(aa_b, ab_b, pa_b, pb_b, da_b, db_b, b2_b, h2_b):
    dinv = _dinv_of(da_b, db_b)
    agg = jnp.concatenate([aa_b[...] + pa_b[...], ab_b[...] + pb_b[...]],
                          axis=1)
    h2_b[...] = jnp.maximum(agg * dinv[:, None] + b2_b[...], 0.0)


def _tc_out_body(sa_b, sb_b, ca_b, cb_b, wc_b, bc_b, o_b):
    sums = sa_b[...] + sb_b[...]
    cnts = ca_b[0, :] + cb_b[0, :]
    pooled = sums[:_G] / jnp.maximum(cnts[:_G], 1.0)[:, None]
    o_b[...] = jnp.dot(pooled, wc_b[...],
                       preferred_element_type=_f32) + bc_b[...]


def _row_spec(w):
    return pl.BlockSpec((_BLK, w), lambda i: (i, 0))


def _full_spec(shape):
    nd = len(shape)
    return pl.BlockSpec(shape, lambda i, _nd=nd: (0,) * _nd)


_deg_spec = pl.BlockSpec((1, _BLK), lambda i: (0, i))


def _tc_g1(h0, dega, degb, W1):
    return pl.pallas_call(
        _tc_g1_body,
        grid=(_GRID,),
        in_specs=[_row_spec(_D), _deg_spec, _deg_spec,
                  _full_spec((_D, _D))],
        out_specs=[_row_spec(_H), _row_spec(_H)],
        out_shape=[jax.ShapeDtypeStruct((_NR, _H), _f32)] * 2,
    )(h0, dega, degb, W1)


def _tc_mid(aa, ab, pa, pb, dega, degb, b1r, W2):
    return pl.pallas_call(
        _tc_mid_body,
        grid=(_GRID,),
        in_specs=[_row_spec(_H), _row_spec(_H), _row_spec(_H), _row_spec(_H),
                  _deg_spec, _deg_spec,
                  _full_spec((1, _D)), _full_spec((_D, _D))],
        out_specs=[_row_spec(_H), _row_spec(_H)],
        out_shape=[jax.ShapeDtypeStruct((_NR, _H), _f32)] * 2,
    )(aa, ab, pa, pb, dega, degb, b1r, W2)


def _tc_h2(aa, ab, pa, pb, dega, degb, b2r):
    return pl.pallas_call(
        _tc_h2_body,
        grid=(_GRID,),
        in_specs=[_row_spec(_H), _row_spec(_H), _row_spec(_H), _row_spec(_H),
                  _deg_spec, _deg_spec, _full_spec((1, _D))],
        out_specs=_row_spec(_D),
        out_shape=jax.ShapeDtypeStruct((_NR, _D), _f32),
    )(aa, ab, pa, pb, dega, degb, b2r)


def _tc_out(sumsa, sumsb, cntsa, cntsb, Wc, bcr):
    return pl.pallas_call(
        _tc_out_body,
        out_shape=jax.ShapeDtypeStruct((_G, 2), _f32),
    )(sumsa, sumsb, cntsa, cntsb, Wc, bcr)


def kernel(x, edge_index, batch, emb, W1, b1, W2, b2, Wc, bc):
    loop = jnp.arange(_N, dtype=_i32)
    src16 = edge_index[0].reshape(_NT, _ECH16, _EC)
    dst16 = edge_index[1].reshape(_NT, _ECH16, _EC)
    dst32 = edge_index[1].reshape(2 * _NT, _ECH32, _EC)

    npad = _NP2 - _N
    xp = jnp.concatenate([x, jnp.zeros((npad,), _i32)]).reshape(
        2 * _NT, _NCH, _EC)
    idp = jnp.concatenate([loop, jnp.zeros((npad,), _i32)]).reshape(
        2 * _NT, _NCH, _EC)
    bp = jnp.concatenate([batch, jnp.full((npad,), _G, _i32)]).reshape(
        2 * _NT, _NCH, _EC)

    z1 = jnp.zeros((_NR,), _f32)
    z2 = jnp.zeros((_NR, _H), _f32)
    b1r = b1.reshape(1, _D)
    b2r = b2.reshape(1, _D)
    bcr = bc.reshape(1, 2)

    h0, dega, degb = _sc_prep(emb, xp, dst32, z1)
    dega = dega.reshape(1, _NR)
    degb = degb.reshape(1, _NR)
    g1a, g1b = _tc_g1(h0, dega, degb, W1)
    a1a, a1b = _sc_edge(g1a, g1b, src16, dst16, z2)
    g2a, g2b = _tc_mid(a1a, a1b, g1a, g1b, dega, degb, b1r, W2)
    a2a, a2b = _sc_edge(g2a, g2b, src16, dst16, z2)
    h2 = _tc_h2(a2a, a2b, g2a, g2b, dega, degb, b2r)
    sumsa, sumsb, cntsa, cntsb = _sc_pool(h2, idp, bp, z1)
    return _tc_out(sumsa, sumsb, cntsa.reshape(1, _GP),
                   cntsb.reshape(1, _GP), Wc, bcr)
